# baseline (speedup 1.0000x reference)
from functools import partial

import jax
import jax.numpy as jnp
from jax.experimental import pallas as pl
from jax.experimental.pallas import tpu as pltpu

_EPS = 1e-07
_DD = 4


def _dice_kernel(x_ref, t_ref, o_ref, acc_i, acc_c, *, c, dd, kb):
    k = pl.program_id(0)

    @pl.when(k == 0)
    def _():
        acc_i[...] = jnp.zeros_like(acc_i)
        acc_c[...] = jnp.zeros_like(acc_c)

    x = x_ref[...]
    t = t_ref[...]
    b = x.shape[0]
    h, w = x.shape[3], x.shape[4]
    prod = (x * t).reshape(b, c * dd, h, w)
    card = (x + t).reshape(b, c * dd, h, w)
    acc_i[...] += jnp.sum(prod, axis=1)
    acc_c[...] += jnp.sum(card, axis=1)

    @pl.when(k == kb - 1)
    def _():
        inter = jnp.sum(jnp.sum(acc_i[...], axis=2), axis=1, keepdims=True)
        card_s = jnp.sum(jnp.sum(acc_c[...], axis=2), axis=1, keepdims=True)
        dice = 1.0 - 2.0 * inter / (card_s + _EPS)
        max_val = jnp.max(dice)
        weights = dice / max_val
        loss = jnp.mean(max_val * weights)
        o_ref[...] = jnp.full(o_ref.shape, loss, jnp.float32)


def kernel(x, target):
    b, c, d, h, w = x.shape

    dd = _DD
    while d % dd:
        dd //= 2
    kb = d // dd

    in_spec = pl.BlockSpec((b, c, dd, h, w), lambda k: (0, 0, k, 0, 0))
    out_spec = pl.BlockSpec((8, 128), lambda k: (0, 0))

    out = pl.pallas_call(
        partial(_dice_kernel, c=c, dd=dd, kb=kb),
        out_shape=jax.ShapeDtypeStruct((8, 128), jnp.float32),
        grid=(kb,),
        in_specs=[in_spec, in_spec],
        out_specs=out_spec,
        scratch_shapes=[pltpu.VMEM((b, h, w), jnp.float32),
                        pltpu.VMEM((b, h, w), jnp.float32)],
        compiler_params=pltpu.CompilerParams(
            vmem_limit_bytes=52 * 1024 * 1024,
        ),
    )(x, target)

    return out[0, 0]
